# baseline (speedup 1.0000x reference)
import jax
import jax.numpy as jnp
from jax.experimental import pallas as pl
from jax.experimental.pallas import tpu as pltpu

_N = 16384
_C = 1000
_BN = 1024


def _lsep_block(yt_ref, yp_ref, out_ref):
    yt = yt_ref[...]
    yp = yp_ref[...]
    is_pos = yt == 1
    sign = jnp.where(is_pos, -1.0, 1.0)
    t = jnp.exp(yp * sign)
    s_neg = jnp.sum(jnp.where(is_pos, 0.0, t), axis=1)
    s_pos = jnp.sum(jnp.where(is_pos, t, 0.0), axis=1)
    block_sum = jnp.sum(jnp.log1p(s_neg * s_pos))

    @pl.when(pl.program_id(0) == 0)
    def _():
        out_ref[0, 0] = 0.0

    out_ref[0, 0] += block_sum


def kernel(y_true, y_pred):
    grid = _N // _BN
    out = pl.pallas_call(
        _lsep_block,
        grid=(grid,),
        in_specs=[
            pl.BlockSpec((_BN, _C), lambda i: (i, 0)),
            pl.BlockSpec((_BN, _C), lambda i: (i, 0)),
        ],
        out_specs=pl.BlockSpec((1, 1), lambda i: (0, 0), memory_space=pltpu.SMEM),
        out_shape=jax.ShapeDtypeStruct((1, 1), jnp.float32),
    )(y_true, y_pred)
    return (out[0, 0] / _N).reshape(1)
